# baseline (speedup 1.0000x reference)
import functools

import jax
import jax.numpy as jnp
from jax.experimental import pallas as pl
from jax.experimental.pallas import tpu as pltpu

MEM_SIZE = 100000
DIM = 256
TOP_K = 32
BATCH = 4096

BQ = 512
BN = 2048
NPAD = 102400


def _sim_kernel(qn_ref, mem_ref, sim_ref):
    j = pl.program_id(1)
    qn = qn_ref[...]
    mem = mem_ref[...]
    sim = jax.lax.dot_general(
        qn, mem, (((1,), (1,)), ((), ())),
        preferred_element_type=jnp.float32)
    col = j * BN + jax.lax.broadcasted_iota(jnp.int32, (BQ, BN), 1)
    sim = jnp.where(col < MEM_SIZE, sim, -jnp.inf)
    sim_ref[...] = sim


def _qn_kernel(q_ref, w_ref, qn_ref):
    q = q_ref[...]
    w = w_ref[...]
    qp = jax.lax.dot_general(
        q, w, (((1,), (1,)), ((), ())), preferred_element_type=jnp.float32)
    norm = jnp.sqrt(jnp.sum(qp * qp, axis=-1, keepdims=True))
    qn_ref[...] = qp / jnp.maximum(norm, 1e-12)


@jax.jit
def kernel(query, memory, W):
    qn = pl.pallas_call(
        _qn_kernel,
        grid=(BATCH // BQ,),
        in_specs=[
            pl.BlockSpec((BQ, DIM), lambda i: (i, 0)),
            pl.BlockSpec((DIM, DIM), lambda i: (0, 0)),
        ],
        out_specs=pl.BlockSpec((BQ, DIM), lambda i: (i, 0)),
        out_shape=jax.ShapeDtypeStruct((BATCH, DIM), jnp.float32),
    )(query, W)

    mem_pad = jnp.pad(memory, ((0, NPAD - MEM_SIZE), (0, 0)))

    sim = pl.pallas_call(
        _sim_kernel,
        grid=(BATCH // BQ, NPAD // BN),
        in_specs=[
            pl.BlockSpec((BQ, DIM), lambda i, j: (i, 0)),
            pl.BlockSpec((BN, DIM), lambda i, j: (j, 0)),
        ],
        out_specs=pl.BlockSpec((BQ, BN), lambda i, j: (i, j)),
        out_shape=jax.ShapeDtypeStruct((BATCH, NPAD), jnp.float32),
    )(qn, mem_pad)

    top_sim, top_idx = jax.lax.top_k(sim, TOP_K)
    weights = jax.nn.softmax(top_sim, axis=-1)
    retrieved = jnp.einsum("bk,bkd->bd", weights, memory[top_idx])
    return retrieved, top_sim


# speedup vs baseline: 7.3730x; 7.3730x over previous
import functools

import jax
import jax.numpy as jnp
from jax import lax
from jax.experimental import pallas as pl
from jax.experimental.pallas import tpu as pltpu
from jax.experimental.pallas import tpu_sc as plsc

MEM_SIZE = 100000
DIM = 256
TOP_K = 32
BATCH = 4096

BQ = 512
BN = 2048
NPAD = 102400
G = 128
NG = NPAD // G
NGJ = BN // G
PAD_VAL = -2.0
INIT_VAL = -3.0

NC = 2
NS = 16
NW = NC * NS
ROWS_PER_W = BATCH // NW


def _qn_kernel(q_ref, w_ref, qn_ref):
    q = q_ref[...]
    w = w_ref[...]
    qp = lax.dot_general(q, w, (((1,), (1,)), ((), ())),
                         preferred_element_type=jnp.float32)
    norm = jnp.sqrt(jnp.sum(qp * qp, axis=-1, keepdims=True))
    qn_ref[...] = qp / jnp.maximum(norm, 1e-12)


def _sim_kernel(qn_ref, mem_ref, sim_ref, gmax_ref):
    j = pl.program_id(1)
    sim = lax.dot_general(qn_ref[...], mem_ref[...], (((1,), (1,)), ((), ())),
                          preferred_element_type=jnp.float32)
    col = j * BN + lax.broadcasted_iota(jnp.int32, (BQ, BN), 1)
    sim = jnp.where(col < MEM_SIZE, sim, PAD_VAL)
    sim_ref[...] = sim
    gmax_ref[...] = jnp.max(sim.reshape(BQ, NGJ, G), axis=2)[None]


def _gmax_t_kernel(g3_ref, g2_ref):
    x = g3_ref[...]
    g2_ref[...] = jnp.transpose(x, (1, 0, 2)).reshape(BQ, NG)


def _combine_kernel(ts_ref, rows_ref, out_ref):
    ts = ts_ref[...]
    rows = rows_ref[...]
    m = jnp.max(ts, axis=-1, keepdims=True)
    e = jnp.exp(ts - m)
    w = e / jnp.sum(e, axis=-1, keepdims=True)
    acc = jnp.zeros((BQ, DIM), jnp.float32)
    for k in range(TOP_K):
        acc = acc + w[:, k][:, None] * rows[:, k, :]
    out_ref[...] = acc


def _iota16():
    return lax.iota(jnp.int32, 16)


_BCAST_DNUMS = lax.GatherDimensionNumbers(
    offset_dims=(), collapsed_slice_dims=(0,), start_index_map=(0,))


def _lane_bcast(vec, lane):
    idx = jnp.full((16, 1), lane, jnp.int32)
    return lax.gather(vec, idx, _BCAST_DNUMS, (1,),
                      mode=lax.GatherScatterMode.PROMISE_IN_BOUNDS)


def _merge_top32(v0, x0, v1, x1, vm, ix):
    s, si = plsc.sort_key_val(vm, ix, descending=True)
    rs = lax.rev(s, (0,))
    rsi = lax.rev(si, (0,))
    take = v0 >= rs
    hi = jnp.where(take, v0, rs)
    hix = jnp.where(take, x0, rsi)
    lo = jnp.where(take, rs, v0)
    lox = jnp.where(take, rsi, x0)
    v0n, x0n = plsc.sort_key_val(hi, hix, descending=True)
    ls, lsi = plsc.sort_key_val(lo, lox, descending=True)
    rls = lax.rev(ls, (0,))
    rlsi = lax.rev(lsi, (0,))
    take2 = v1 >= rls
    hi2 = jnp.where(take2, v1, rls)
    hix2 = jnp.where(take2, x1, rlsi)
    v1n, x1n = plsc.sort_key_val(hi2, hix2, descending=True)
    return v0n, x0n, v1n, x1n


def _maybe_merge(state, v, ix):
    v0, x0, v1, x1 = state
    m = v > _lane_bcast(v1, 15)
    return lax.cond(
        jnp.any(m),
        lambda: _merge_top32(v0, x0, v1, x1, jnp.where(m, v, INIT_VAL), ix),
        lambda: state,
    )


def _init_state():
    z = jnp.full((16,), INIT_VAL, jnp.float32)
    zi = jnp.zeros((16,), jnp.int32)
    return (z, zi, z, zi)


def _make_sc_kernel():
    mesh = plsc.VectorSubcoreMesh(core_axis_name="c", subcore_axis_name="s")

    @functools.partial(
        pl.kernel,
        mesh=mesh,
        out_type=[
            jax.ShapeDtypeStruct((BATCH, TOP_K), jnp.float32),
            jax.ShapeDtypeStruct((BATCH, TOP_K, DIM), jnp.float32),
        ],
        scratch_types=[
            pltpu.VMEM((NG,), jnp.float32),
            pltpu.VMEM((TOP_K,), jnp.int32),
            pltpu.VMEM((TOP_K,), jnp.int32),
            pltpu.VMEM((TOP_K, G), jnp.float32),
            pltpu.VMEM((TOP_K, G), jnp.int32),
            pltpu.VMEM((TOP_K, DIM), jnp.float32),
            pltpu.VMEM((TOP_K,), jnp.float32),
            pltpu.SemaphoreType.DMA,
        ],
        compiler_params=pltpu.CompilerParams(needs_layout_passes=False),
    )
    def sc_kernel(simc_hbm, gmax_hbm, colc_hbm, memory_hbm,
                  tsim_hbm, rows_hbm,
                  gmax_v, gid_v, idx_v, cand_v, col_v, rows_v, ts_v, sem):
        wid = lax.axis_index("s") * NC + lax.axis_index("c")
        row0 = wid * ROWS_PER_W

        def row_body(rr, _):
            r = row0 + rr
            pltpu.sync_copy(gmax_hbm.at[r], gmax_v)

            def s1_body(j, state):
                v = gmax_v[pl.ds(j * 16, 16)]
                ix = j * 16 + _iota16()
                return _maybe_merge(state, v, ix)

            _, gx0, _, gx1 = lax.fori_loop(0, NG // 16, s1_body, _init_state())

            gid_v[pl.ds(0, 16)] = gx0
            gid_v[pl.ds(16, 16)] = gx1
            idx_v[pl.ds(0, 16)] = gx0 + r * NG
            idx_v[pl.ds(16, 16)] = gx1 + r * NG
            pltpu.async_copy(simc_hbm.at[idx_v], cand_v, sem).wait()
            pltpu.async_copy(colc_hbm.at[gid_v], col_v, sem).wait()

            state = _init_state()
            for g in range(TOP_K):
                def s2_body(j, st, g=g):
                    v = cand_v[g, pl.ds(j * 16, 16)]
                    ix = col_v[g, pl.ds(j * 16, 16)]
                    return _maybe_merge(st, v, ix)

                state = lax.fori_loop(0, G // 16, s2_body, state)
            v0, x0, v1, x1 = state

            ts_v[pl.ds(0, 16)] = v0
            ts_v[pl.ds(16, 16)] = v1
            pltpu.sync_copy(ts_v, tsim_hbm.at[r])

            idx_v[pl.ds(0, 16)] = x0
            idx_v[pl.ds(16, 16)] = x1
            pltpu.async_copy(memory_hbm.at[idx_v], rows_v, sem).wait()
            pltpu.sync_copy(rows_v, rows_hbm.at[r])
            return 0

        lax.fori_loop(0, ROWS_PER_W, row_body, 0)

    return sc_kernel


_sc_kernel = _make_sc_kernel()


@jax.jit
def kernel(query, memory, W):
    qn = pl.pallas_call(
        _qn_kernel,
        grid=(BATCH // BQ,),
        in_specs=[
            pl.BlockSpec((BQ, DIM), lambda i: (i, 0)),
            pl.BlockSpec((DIM, DIM), lambda i: (0, 0)),
        ],
        out_specs=pl.BlockSpec((BQ, DIM), lambda i: (i, 0)),
        out_shape=jax.ShapeDtypeStruct((BATCH, DIM), jnp.float32),
    )(query, W)

    mem_pad = jnp.pad(memory, ((0, NPAD - MEM_SIZE), (0, 0)))

    sim, gmax3 = pl.pallas_call(
        _sim_kernel,
        grid=(BATCH // BQ, NPAD // BN),
        in_specs=[
            pl.BlockSpec((BQ, DIM), lambda i, j: (i, 0)),
            pl.BlockSpec((BN, DIM), lambda i, j: (j, 0)),
        ],
        out_specs=[
            pl.BlockSpec((BQ, BN), lambda i, j: (i, j)),
            pl.BlockSpec((1, BQ, NGJ), lambda i, j: (j, i, 0)),
        ],
        out_shape=[
            jax.ShapeDtypeStruct((BATCH, NPAD), jnp.float32),
            jax.ShapeDtypeStruct((NPAD // BN, BATCH, NGJ), jnp.float32),
        ],
    )(qn, mem_pad)

    gmax = pl.pallas_call(
        _gmax_t_kernel,
        grid=(BATCH // BQ,),
        in_specs=[pl.BlockSpec((NPAD // BN, BQ, NGJ), lambda i: (0, i, 0))],
        out_specs=pl.BlockSpec((BQ, NG), lambda i: (i, 0)),
        out_shape=jax.ShapeDtypeStruct((BATCH, NG), jnp.float32),
    )(gmax3)

    sim_chunks = sim.reshape(BATCH * NG, G)
    col_chunks = jnp.arange(NG * G, dtype=jnp.int32).reshape(NG, G)

    top_sim, rows = _sc_kernel(sim_chunks, gmax, col_chunks, memory)

    retrieved = pl.pallas_call(
        _combine_kernel,
        grid=(BATCH // BQ,),
        in_specs=[
            pl.BlockSpec((BQ, TOP_K), lambda i: (i, 0)),
            pl.BlockSpec((BQ, TOP_K, DIM), lambda i: (i, 0, 0)),
        ],
        out_specs=pl.BlockSpec((BQ, DIM), lambda i: (i, 0)),
        out_shape=jax.ShapeDtypeStruct((BATCH, DIM), jnp.float32),
    )(top_sim, rows)

    return retrieved, top_sim


# speedup vs baseline: 7.9396x; 1.0768x over previous
import functools

import jax
import jax.numpy as jnp
from jax import lax
from jax.experimental import pallas as pl
from jax.experimental.pallas import tpu as pltpu
from jax.experimental.pallas import tpu_sc as plsc

MEM_SIZE = 100000
DIM = 256
TOP_K = 32
BATCH = 4096

BQ = 512
BN = 2048
NPAD = 102400
G = 128
NG = NPAD // G
NGJ = BN // G
PAD_VAL = -2.0
INIT_VAL = -3.0

NC = 2
NS = 16
NW = NC * NS
ROWS_PER_W = BATCH // NW


def _qn_kernel(q_ref, w_ref, qn_ref):
    q = q_ref[...]
    w = w_ref[...]
    qp = lax.dot_general(q, w, (((1,), (1,)), ((), ())),
                         preferred_element_type=jnp.float32)
    norm = jnp.sqrt(jnp.sum(qp * qp, axis=-1, keepdims=True))
    qn_ref[...] = qp / jnp.maximum(norm, 1e-12)


def _sim_kernel(qn_ref, mem_ref, sim_ref, gmax_ref):
    j = pl.program_id(1)
    sim = lax.dot_general(qn_ref[...], mem_ref[...], (((1,), (1,)), ((), ())),
                          preferred_element_type=jnp.float32)
    col = j * BN + lax.broadcasted_iota(jnp.int32, (BQ, BN), 1)
    sim = jnp.where(col < MEM_SIZE, sim, PAD_VAL)
    sim_ref[...] = sim
    gmax_ref[...] = jnp.max(sim.reshape(BQ, NGJ, G), axis=2)[None]


def _gmax_t_kernel(g3_ref, g2_ref):
    x = g3_ref[...]
    g2_ref[...] = jnp.transpose(x, (1, 0, 2)).reshape(BQ, NG)


def _combine_kernel(ts_ref, rows_ref, out_ref):
    ts = ts_ref[...]
    rows = rows_ref[...]
    m = jnp.max(ts, axis=-1, keepdims=True)
    e = jnp.exp(ts - m)
    w = e / jnp.sum(e, axis=-1, keepdims=True)
    acc = jnp.zeros((BQ, DIM), jnp.float32)
    for k in range(TOP_K):
        acc = acc + w[:, k][:, None] * rows[:, k, :]
    out_ref[...] = acc


def _iota16():
    return lax.iota(jnp.int32, 16)


_BCAST_DNUMS = lax.GatherDimensionNumbers(
    offset_dims=(), collapsed_slice_dims=(0,), start_index_map=(0,))


def _lane_bcast(vec, lane):
    idx = jnp.full((16, 1), lane, jnp.int32)
    return lax.gather(vec, idx, _BCAST_DNUMS, (1,),
                      mode=lax.GatherScatterMode.PROMISE_IN_BOUNDS)


def _merge_top32(v0, x0, v1, x1, vm, ix):
    s, si = plsc.sort_key_val(vm, ix, descending=True)
    rs = lax.rev(s, (0,))
    rsi = lax.rev(si, (0,))
    take = v0 >= rs
    hi = jnp.where(take, v0, rs)
    hix = jnp.where(take, x0, rsi)
    lo = jnp.where(take, rs, v0)
    lox = jnp.where(take, rsi, x0)
    v0n, x0n = plsc.sort_key_val(hi, hix, descending=True)
    ls, lsi = plsc.sort_key_val(lo, lox, descending=True)
    rls = lax.rev(ls, (0,))
    rlsi = lax.rev(lsi, (0,))
    take2 = v1 >= rls
    hi2 = jnp.where(take2, v1, rls)
    hix2 = jnp.where(take2, x1, rlsi)
    v1n, x1n = plsc.sort_key_val(hi2, hix2, descending=True)
    return v0n, x0n, v1n, x1n


def _maybe_merge(state, v, ix, th0=None):
    v0, x0, v1, x1 = state
    m = v > _lane_bcast(v1, 15)
    if th0 is not None:
        m = m | (v >= th0)
    return lax.cond(
        jnp.any(m),
        lambda: _merge_top32(v0, x0, v1, x1, jnp.where(m, v, INIT_VAL), ix),
        lambda: state,
    )


def _init_state():
    z = jnp.full((16,), INIT_VAL, jnp.float32)
    zi = jnp.zeros((16,), jnp.int32)
    return (z, zi, z, zi)


R = 4
NCHW = ROWS_PER_W // R


def _make_sc_kernel():
    mesh = plsc.VectorSubcoreMesh(core_axis_name="c", subcore_axis_name="s")

    @functools.partial(
        pl.kernel,
        mesh=mesh,
        out_type=[
            jax.ShapeDtypeStruct((BATCH * TOP_K,), jnp.float32),
            jax.ShapeDtypeStruct((BATCH * TOP_K, DIM), jnp.float32),
        ],
        scratch_types=[
            pltpu.VMEM((2, R * NG), jnp.float32),
            pltpu.VMEM((R * TOP_K,), jnp.int32),
            pltpu.VMEM((R * TOP_K, G), jnp.float32),
            pltpu.VMEM((R * TOP_K, DIM), jnp.float32),
            pltpu.VMEM((R * TOP_K,), jnp.float32),
            pltpu.SemaphoreType.DMA,
            pltpu.SemaphoreType.DMA,
            pltpu.SemaphoreType.DMA,
        ],
        compiler_params=pltpu.CompilerParams(needs_layout_passes=False),
    )
    def sc_kernel(simc_hbm, gmax_hbm, memory_hbm,
                  tsim_hbm, rows_hbm,
                  gmax_v, idx_v, cand_v, rows_v, ts_v, sem_g, sem_c, sem_r):
        wid = lax.axis_index("s") * NC + lax.axis_index("c")
        row0 = wid * ROWS_PER_W

        pltpu.make_async_copy(
            gmax_hbm.at[pl.ds(row0 * NG, R * NG)], gmax_v.at[0], sem_g).start()

        def chunk_body(chunk, b):
            grow0 = row0 + chunk * R
            pltpu.make_async_copy(
                gmax_hbm.at[pl.ds(0, R * NG)], gmax_v.at[b], sem_g).wait()
            nxt = row0 + jnp.minimum(chunk + 1, NCHW - 1) * R
            pltpu.make_async_copy(
                gmax_hbm.at[pl.ds(nxt * NG, R * NG)],
                gmax_v.at[1 - b], sem_g).start()

            s1 = []
            for rr in range(R):
                def s1_body(j, state, rr=rr):
                    v = gmax_v[b, pl.ds(rr * NG + j * 16, 16)]
                    ix = j * 16 + _iota16()
                    return _maybe_merge(state, v, ix)

                _, gx0, v1g, gx1 = lax.fori_loop(
                    0, NG // 16, s1_body, _init_state())
                s1.append((gx0, gx1, v1g))
                idx_v[pl.ds(rr * TOP_K, 16)] = gx0 + (grow0 + rr) * NG
                idx_v[pl.ds(rr * TOP_K + 16, 16)] = gx1 + (grow0 + rr) * NG

            pltpu.make_async_copy(simc_hbm.at[idx_v], cand_v, sem_c).start()
            pltpu.make_async_copy(simc_hbm.at[idx_v], cand_v, sem_c).wait()

            for rr in range(R):
                gx0, gx1, v1g = s1[rr]
                th0 = _lane_bcast(v1g, 15)

                def s2g(g, state, rr=rr, gx0=gx0, gx1=gx1, th0=th0):
                    civ = jnp.where(
                        g < 16,
                        _lane_bcast(gx0, g),
                        _lane_bcast(gx1, jnp.maximum(g - 16, 0)))
                    base = civ * G

                    def s2j(j, st):
                        v = cand_v[rr * TOP_K + g, pl.ds(j * 16, 16)]
                        ix = base + j * 16 + _iota16()
                        return _maybe_merge(st, v, ix, th0=th0)

                    return lax.fori_loop(0, G // 16, s2j, state)

                v0, x0, v1, x1 = lax.fori_loop(0, TOP_K, s2g, _init_state())
                ts_v[pl.ds(rr * TOP_K, 16)] = v0
                ts_v[pl.ds(rr * TOP_K + 16, 16)] = v1
                idx_v[pl.ds(rr * TOP_K, 16)] = x0
                idx_v[pl.ds(rr * TOP_K + 16, 16)] = x1

            pltpu.make_async_copy(memory_hbm.at[idx_v], rows_v, sem_r).start()
            pltpu.sync_copy(ts_v, tsim_hbm.at[pl.ds(grow0 * TOP_K, R * TOP_K)])
            pltpu.make_async_copy(memory_hbm.at[idx_v], rows_v, sem_r).wait()
            pltpu.sync_copy(rows_v,
                            rows_hbm.at[pl.ds(grow0 * TOP_K, R * TOP_K)])

        def cc_body(cc, _):
            chunk_body(cc * 2, 0)
            chunk_body(cc * 2 + 1, 1)
            return 0

        lax.fori_loop(0, NCHW // 2, cc_body, 0)
        pltpu.make_async_copy(
            gmax_hbm.at[pl.ds(0, R * NG)], gmax_v.at[0], sem_g).wait()

    return sc_kernel


_sc_kernel = _make_sc_kernel()


@jax.jit
def kernel(query, memory, W):
    qn = pl.pallas_call(
        _qn_kernel,
        grid=(BATCH // BQ,),
        in_specs=[
            pl.BlockSpec((BQ, DIM), lambda i: (i, 0)),
            pl.BlockSpec((DIM, DIM), lambda i: (0, 0)),
        ],
        out_specs=pl.BlockSpec((BQ, DIM), lambda i: (i, 0)),
        out_shape=jax.ShapeDtypeStruct((BATCH, DIM), jnp.float32),
    )(query, W)

    mem_pad = jnp.pad(memory, ((0, NPAD - MEM_SIZE), (0, 0)))

    sim, gmax3 = pl.pallas_call(
        _sim_kernel,
        grid=(BATCH // BQ, NPAD // BN),
        in_specs=[
            pl.BlockSpec((BQ, DIM), lambda i, j: (i, 0)),
            pl.BlockSpec((BN, DIM), lambda i, j: (j, 0)),
        ],
        out_specs=[
            pl.BlockSpec((BQ, BN), lambda i, j: (i, j)),
            pl.BlockSpec((1, BQ, NGJ), lambda i, j: (j, i, 0)),
        ],
        out_shape=[
            jax.ShapeDtypeStruct((BATCH, NPAD), jnp.float32),
            jax.ShapeDtypeStruct((NPAD // BN, BATCH, NGJ), jnp.float32),
        ],
    )(qn, mem_pad)

    gmax = pl.pallas_call(
        _gmax_t_kernel,
        grid=(BATCH // BQ,),
        in_specs=[pl.BlockSpec((NPAD // BN, BQ, NGJ), lambda i: (0, i, 0))],
        out_specs=pl.BlockSpec((BQ, NG), lambda i: (i, 0)),
        out_shape=jax.ShapeDtypeStruct((BATCH, NG), jnp.float32),
    )(gmax3)

    sim_chunks = sim.reshape(BATCH * NG, G)

    top_sim_f, rows_f = _sc_kernel(sim_chunks, gmax.reshape(-1), memory)
    top_sim = top_sim_f.reshape(BATCH, TOP_K)
    rows = rows_f.reshape(BATCH, TOP_K, DIM)

    retrieved = pl.pallas_call(
        _combine_kernel,
        grid=(BATCH // BQ,),
        in_specs=[
            pl.BlockSpec((BQ, TOP_K), lambda i: (i, 0)),
            pl.BlockSpec((BQ, TOP_K, DIM), lambda i: (i, 0, 0)),
        ],
        out_specs=pl.BlockSpec((BQ, DIM), lambda i: (i, 0)),
        out_shape=jax.ShapeDtypeStruct((BATCH, DIM), jnp.float32),
    )(top_sim, rows)

    return retrieved, top_sim


# speedup vs baseline: 8.6809x; 1.0934x over previous
import functools

import jax
import jax.numpy as jnp
from jax import lax
from jax.experimental import pallas as pl
from jax.experimental.pallas import tpu as pltpu
from jax.experimental.pallas import tpu_sc as plsc

MEM_SIZE = 100000
DIM = 256
TOP_K = 32
BATCH = 4096

BQ = 512
BN = 2048
NPAD = 102400
G = 128
NG = NPAD // G
NGJ = BN // G
PAD_VAL = -2.0
INIT_VAL = -3.0

NC = 2
NS = 16
NW = NC * NS
ROWS_PER_W = BATCH // NW


def _qn_kernel(q_ref, w_ref, qn_ref):
    q = q_ref[...]
    w = w_ref[...]
    qp = lax.dot_general(q, w, (((1,), (1,)), ((), ())),
                         preferred_element_type=jnp.float32)
    norm = jnp.sqrt(jnp.sum(qp * qp, axis=-1, keepdims=True))
    qn_ref[...] = qp / jnp.maximum(norm, 1e-12)


def _sim_kernel(qn_ref, mem_ref, sim_ref, gmax_ref):
    j = pl.program_id(1)
    sim = lax.dot_general(qn_ref[...], mem_ref[...], (((1,), (1,)), ((), ())),
                          preferred_element_type=jnp.float32)
    col = j * BN + lax.broadcasted_iota(jnp.int32, (BQ, BN), 1)
    sim = jnp.where(col < MEM_SIZE, sim, PAD_VAL)
    sim_ref[...] = sim
    gmax_ref[...] = jnp.max(sim.reshape(BQ, NGJ, G), axis=2)[None]


def _gmax_t_kernel(g3_ref, g2_ref):
    x = g3_ref[...]
    g2_ref[...] = jnp.transpose(x, (1, 0, 2)).reshape(BQ, NG)


def _combine_kernel(ts_ref, rows_ref, out_ref):
    ts = ts_ref[...]
    rows = rows_ref[...]
    m = jnp.max(ts, axis=-1, keepdims=True)
    e = jnp.exp(ts - m)
    w = e / jnp.sum(e, axis=-1, keepdims=True)
    acc = jnp.zeros((BQ, DIM), jnp.float32)
    for k in range(TOP_K):
        acc = acc + w[:, k][:, None] * rows[:, k, :]
    out_ref[...] = acc


def _iota16():
    return lax.iota(jnp.int32, 16)


_BCAST_DNUMS = lax.GatherDimensionNumbers(
    offset_dims=(), collapsed_slice_dims=(0,), start_index_map=(0,))


def _lane_bcast(vec, lane):
    idx = jnp.full((16, 1), lane, jnp.int32)
    return lax.gather(vec, idx, _BCAST_DNUMS, (1,),
                      mode=lax.GatherScatterMode.PROMISE_IN_BOUNDS)


def _merge_top32(v0, x0, v1, x1, vm, ix):
    s, si = plsc.sort_key_val(vm, ix, descending=True)
    rs = lax.rev(s, (0,))
    rsi = lax.rev(si, (0,))
    take = v0 >= rs
    hi = jnp.where(take, v0, rs)
    hix = jnp.where(take, x0, rsi)
    lo = jnp.where(take, rs, v0)
    lox = jnp.where(take, rsi, x0)
    v0n, x0n = plsc.sort_key_val(hi, hix, descending=True)
    ls, lsi = plsc.sort_key_val(lo, lox, descending=True)
    rls = lax.rev(ls, (0,))
    rlsi = lax.rev(lsi, (0,))
    take2 = v1 >= rls
    hi2 = jnp.where(take2, v1, rls)
    hix2 = jnp.where(take2, x1, rlsi)
    v1n, x1n = plsc.sort_key_val(hi2, hix2, descending=True)
    return v0n, x0n, v1n, x1n


def _maybe_merge(state, v, ix, th0=None):
    v0, x0, v1, x1 = state
    m = v > _lane_bcast(v1, 15)
    if th0 is not None:
        m = m | (v >= th0)
    return lax.cond(
        jnp.any(m),
        lambda: _merge_top32(v0, x0, v1, x1, jnp.where(m, v, INIT_VAL), ix),
        lambda: state,
    )


def _init_state():
    z = jnp.full((16,), INIT_VAL, jnp.float32)
    zi = jnp.zeros((16,), jnp.int32)
    return (z, zi, z, zi)


R = 2
C = 64
C2 = 160
BIS_IT = 22
NCHW = ROWS_PER_W // R
GSTRIDE = R * NG + 64


def _make_sc_kernel():
    mesh = plsc.VectorSubcoreMesh(core_axis_name="c", subcore_axis_name="s")

    @functools.partial(
        pl.kernel,
        mesh=mesh,
        out_type=[
            jax.ShapeDtypeStruct((BATCH * TOP_K,), jnp.float32),
            jax.ShapeDtypeStruct((BATCH * TOP_K, DIM), jnp.float32),
        ],
        scratch_types=[
            pltpu.VMEM((GSTRIDE,), jnp.float32),
            pltpu.VMEM((GSTRIDE,), jnp.float32),
            pltpu.VMEM((R * C,), jnp.int32),
            pltpu.VMEM((R * C, G), jnp.float32),
            pltpu.VMEM((C2,), jnp.float32),
            pltpu.VMEM((C2,), jnp.int32),
            pltpu.VMEM((R * TOP_K,), jnp.int32),
            pltpu.VMEM((R * TOP_K, DIM), jnp.float32),
            pltpu.VMEM((R * TOP_K,), jnp.float32),
            pltpu.SemaphoreType.DMA,
            pltpu.SemaphoreType.DMA,
            pltpu.SemaphoreType.DMA,
        ],
        compiler_params=pltpu.CompilerParams(needs_layout_passes=False),
    )
    def sc_kernel(simc_hbm, gmax_hbm, memory_hbm,
                  tsim_hbm, rows_hbm,
                  gmax_a, gmax_b, idx_v, cand_v, c2v, c2i, slot_v, rows_v,
                  ts_v, sem_g, sem_c, sem_r):
        gmax_bufs = (gmax_a, gmax_b)
        wid = lax.axis_index("s") * NC + lax.axis_index("c")
        row0 = wid * ROWS_PER_W

        pltpu.make_async_copy(
            gmax_hbm.at[pl.ds(row0 * NG, GSTRIDE)], gmax_a, sem_g).start()

        def chunk_body(chunk, b):
            grow0 = row0 + chunk * R
            pltpu.make_async_copy(
                gmax_hbm.at[pl.ds(0, GSTRIDE)], gmax_bufs[b], sem_g).wait()
            nxt = row0 + jnp.minimum(chunk + 1, NCHW - 1) * R
            pltpu.make_async_copy(
                gmax_hbm.at[pl.ds(nxt * NG, GSTRIDE)],
                gmax_bufs[1 - b], sem_g).start()

            def s1_row(rr, ths):
                def count_gt(t):
                    def cnt_body(j, acc):
                        v = gmax_bufs[b][pl.ds(rr * NG + j * 16, 16)]
                        return acc + plsc.all_reduce_population_count(v > t)
                    return lax.fori_loop(
                        0, NG // 16, cnt_body,
                        jnp.zeros((16,), jnp.int32))[0]

                def bis_body(i, lh):
                    lo, hi = lh
                    mid = 0.5 * (lo + hi)
                    big = count_gt(mid) >= TOP_K
                    return (jnp.where(big, mid, lo), jnp.where(big, hi, mid))

                th, _ = lax.fori_loop(
                    0, BIS_IT, bis_body,
                    (jnp.float32(-1.0), jnp.float32(1.0)))

                pad_chunk = jnp.full((16,), (grow0 + rr) * NG + NG - 1,
                                     jnp.int32)
                for q in range(C // 16):
                    idx_v[pl.ds(rr * C + q * 16, 16)] = pad_chunk

                def col_body(j, cur):
                    v = gmax_bufs[b][pl.ds(rr * NG + j * 16, 16)]
                    m = v > th
                    gid = j * 16 + _iota16() + (grow0 + rr) * NG
                    plsc.store_compressed(idx_v.at[pl.ds(cur, 16)], gid,
                                          mask=m)
                    cnt = plsc.all_reduce_population_count(m)[0]
                    return jnp.minimum(cur + cnt, rr * C + (C - 16))

                lax.fori_loop(0, NG // 16, col_body, rr * C)
                return ths + (th,)

            ths = ()
            for rr in range(R):
                ths = s1_row(rr, ths)

            pltpu.make_async_copy(simc_hbm.at[idx_v], cand_v, sem_c).start()
            pltpu.make_async_copy(simc_hbm.at[idx_v], cand_v, sem_c).wait()

            for rr in range(R):
                th = ths[rr]
                neg = jnp.full((16,), INIT_VAL, jnp.float32)
                for q in range(C2 // 16):
                    c2v[pl.ds(q * 16, 16)] = neg

                def scan_body(t, cur, rr=rr, th=th):
                    civ16 = idx_v[pl.ds(rr * C + (t >> 4) * 16, 16)]
                    civ = _lane_bcast(civ16, t & 15)
                    colbase = (civ - (grow0 + rr) * NG) * G
                    for j in range(G // 16):
                        v = cand_v[rr * C + t, pl.ds(j * 16, 16)]
                        m = v > th
                        col = colbase + j * 16 + _iota16()
                        plsc.store_compressed(c2v.at[pl.ds(cur, 16)], v,
                                              mask=m)
                        plsc.store_compressed(c2i.at[pl.ds(cur, 16)], col,
                                              mask=m)
                        cnt = plsc.all_reduce_population_count(m)[0]
                        cur = jnp.minimum(cur + cnt, C2 - 16)
                    return cur

                lax.fori_loop(0, C, scan_body, 0)

                state = _init_state()
                for q in range(C2 // 16):
                    state = _merge_top32(*state,
                                         c2v[pl.ds(q * 16, 16)],
                                         c2i[pl.ds(q * 16, 16)])
                v0, x0, v1, x1 = state
                ts_v[pl.ds(rr * TOP_K, 16)] = v0
                ts_v[pl.ds(rr * TOP_K + 16, 16)] = v1
                slot_v[pl.ds(rr * TOP_K, 16)] = x0
                slot_v[pl.ds(rr * TOP_K + 16, 16)] = x1

            pltpu.make_async_copy(memory_hbm.at[slot_v], rows_v, sem_r).start()
            pltpu.sync_copy(ts_v, tsim_hbm.at[pl.ds(grow0 * TOP_K, R * TOP_K)])
            pltpu.make_async_copy(memory_hbm.at[slot_v], rows_v, sem_r).wait()
            pltpu.sync_copy(rows_v,
                            rows_hbm.at[pl.ds(grow0 * TOP_K, R * TOP_K)])

        def cc_body(cc, _):
            chunk_body(cc * 2, 0)
            chunk_body(cc * 2 + 1, 1)
            return 0

        lax.fori_loop(0, NCHW // 2, cc_body, 0)
        pltpu.make_async_copy(
            gmax_hbm.at[pl.ds(0, GSTRIDE)], gmax_a, sem_g).wait()

    return sc_kernel


_sc_kernel = _make_sc_kernel()


@jax.jit
def kernel(query, memory, W):
    qn = pl.pallas_call(
        _qn_kernel,
        grid=(BATCH // BQ,),
        in_specs=[
            pl.BlockSpec((BQ, DIM), lambda i: (i, 0)),
            pl.BlockSpec((DIM, DIM), lambda i: (0, 0)),
        ],
        out_specs=pl.BlockSpec((BQ, DIM), lambda i: (i, 0)),
        out_shape=jax.ShapeDtypeStruct((BATCH, DIM), jnp.float32),
    )(query, W)

    mem_pad = jnp.pad(memory, ((0, NPAD - MEM_SIZE), (0, 0)))

    sim, gmax3 = pl.pallas_call(
        _sim_kernel,
        grid=(BATCH // BQ, NPAD // BN),
        in_specs=[
            pl.BlockSpec((BQ, DIM), lambda i, j: (i, 0)),
            pl.BlockSpec((BN, DIM), lambda i, j: (j, 0)),
        ],
        out_specs=[
            pl.BlockSpec((BQ, BN), lambda i, j: (i, j)),
            pl.BlockSpec((1, BQ, NGJ), lambda i, j: (j, i, 0)),
        ],
        out_shape=[
            jax.ShapeDtypeStruct((BATCH, NPAD), jnp.float32),
            jax.ShapeDtypeStruct((NPAD // BN, BATCH, NGJ), jnp.float32),
        ],
    )(qn, mem_pad)

    gmax = pl.pallas_call(
        _gmax_t_kernel,
        grid=(BATCH // BQ,),
        in_specs=[pl.BlockSpec((NPAD // BN, BQ, NGJ), lambda i: (0, i, 0))],
        out_specs=pl.BlockSpec((BQ, NG), lambda i: (i, 0)),
        out_shape=jax.ShapeDtypeStruct((BATCH, NG), jnp.float32),
    )(gmax3)

    sim_chunks = sim.reshape(BATCH * NG, G)

    gmax_flat = jnp.pad(gmax.reshape(-1), (0, 128))
    top_sim_f, rows_f = _sc_kernel(sim_chunks, gmax_flat, memory)
    top_sim = top_sim_f.reshape(BATCH, TOP_K)
    rows = rows_f.reshape(BATCH, TOP_K, DIM)

    retrieved = pl.pallas_call(
        _combine_kernel,
        grid=(BATCH // BQ,),
        in_specs=[
            pl.BlockSpec((BQ, TOP_K), lambda i: (i, 0)),
            pl.BlockSpec((BQ, TOP_K, DIM), lambda i: (i, 0, 0)),
        ],
        out_specs=pl.BlockSpec((BQ, DIM), lambda i: (i, 0)),
        out_shape=jax.ShapeDtypeStruct((BATCH, DIM), jnp.float32),
    )(top_sim, rows)

    return retrieved, top_sim


# speedup vs baseline: 10.1217x; 1.1660x over previous
import functools

import jax
import jax.numpy as jnp
from jax import lax
from jax.experimental import pallas as pl
from jax.experimental.pallas import tpu as pltpu
from jax.experimental.pallas import tpu_sc as plsc

MEM_SIZE = 100000
DIM = 256
TOP_K = 32
BATCH = 4096

BQ = 512
BN = 2048
NPAD = 102400
G = 128
NG = NPAD // G
NGJ = BN // G
PAD_VAL = -2.0
INIT_VAL = -3.0

NC = 2
NS = 16
NW = NC * NS
ROWS_PER_W = BATCH // NW


def _qn_kernel(q_ref, w_ref, qn_ref):
    q = q_ref[...]
    w = w_ref[...]
    qp = lax.dot_general(q, w, (((1,), (1,)), ((), ())),
                         preferred_element_type=jnp.float32)
    norm = jnp.sqrt(jnp.sum(qp * qp, axis=-1, keepdims=True))
    qn_ref[...] = qp / jnp.maximum(norm, 1e-12)


def _sim_kernel(qn_ref, mem_ref, sim_ref, gmax_ref):
    j = pl.program_id(1)
    sim = lax.dot_general(qn_ref[...], mem_ref[...], (((1,), (1,)), ((), ())),
                          preferred_element_type=jnp.float32)
    col = j * BN + lax.broadcasted_iota(jnp.int32, (BQ, BN), 1)
    sim = jnp.where(col < MEM_SIZE, sim, PAD_VAL)
    sim_ref[...] = sim
    gmax_ref[...] = jnp.max(sim.reshape(BQ, NGJ, G), axis=2)[None]


def _gmax_t_kernel(g3_ref, g2_ref):
    x = g3_ref[...]
    g2_ref[...] = jnp.transpose(x, (1, 0, 2)).reshape(BQ, NG)


def _combine_kernel(ts_ref, rows_ref, out_ref):
    ts = ts_ref[...]
    rows = rows_ref[...]
    m = jnp.max(ts, axis=-1, keepdims=True)
    e = jnp.exp(ts - m)
    w = e / jnp.sum(e, axis=-1, keepdims=True)
    acc = jnp.zeros((BQ, DIM), jnp.float32)
    for k in range(TOP_K):
        acc = acc + w[:, k][:, None] * rows[:, k, :]
    out_ref[...] = acc


def _iota16():
    return lax.iota(jnp.int32, 16)


_BCAST_DNUMS = lax.GatherDimensionNumbers(
    offset_dims=(), collapsed_slice_dims=(0,), start_index_map=(0,))


def _lane_bcast(vec, lane):
    idx = jnp.full((16, 1), lane, jnp.int32)
    return lax.gather(vec, idx, _BCAST_DNUMS, (1,),
                      mode=lax.GatherScatterMode.PROMISE_IN_BOUNDS)


def _merge_top32(v0, x0, v1, x1, vm, ix):
    s, si = plsc.sort_key_val(vm, ix, descending=True)
    rs = lax.rev(s, (0,))
    rsi = lax.rev(si, (0,))
    take = v0 >= rs
    hi = jnp.where(take, v0, rs)
    hix = jnp.where(take, x0, rsi)
    lo = jnp.where(take, rs, v0)
    lox = jnp.where(take, rsi, x0)
    v0n, x0n = plsc.sort_key_val(hi, hix, descending=True)
    ls, lsi = plsc.sort_key_val(lo, lox, descending=True)
    rls = lax.rev(ls, (0,))
    rlsi = lax.rev(lsi, (0,))
    take2 = v1 >= rls
    hi2 = jnp.where(take2, v1, rls)
    hix2 = jnp.where(take2, x1, rlsi)
    v1n, x1n = plsc.sort_key_val(hi2, hix2, descending=True)
    return v0n, x0n, v1n, x1n


def _maybe_merge(state, v, ix, th0=None):
    v0, x0, v1, x1 = state
    m = v > _lane_bcast(v1, 15)
    if th0 is not None:
        m = m | (v >= th0)
    return lax.cond(
        jnp.any(m),
        lambda: _merge_top32(v0, x0, v1, x1, jnp.where(m, v, INIT_VAL), ix),
        lambda: state,
    )


def _init_state():
    z = jnp.full((16,), INIT_VAL, jnp.float32)
    zi = jnp.zeros((16,), jnp.int32)
    return (z, zi, z, zi)


R = 2
C = 64
C2 = 160
BIS_IT = 12
NCHW = ROWS_PER_W // R
GSTRIDE = R * NG + 64


def _make_sc_kernel():
    mesh = plsc.VectorSubcoreMesh(core_axis_name="c", subcore_axis_name="s")

    @functools.partial(
        pl.kernel,
        mesh=mesh,
        out_type=[
            jax.ShapeDtypeStruct((BATCH * TOP_K,), jnp.float32),
            jax.ShapeDtypeStruct((BATCH * TOP_K, DIM), jnp.float32),
        ],
        scratch_types=[
            pltpu.VMEM((GSTRIDE,), jnp.float32),
            pltpu.VMEM((GSTRIDE,), jnp.float32),
            pltpu.VMEM((R * C,), jnp.int32),
            pltpu.VMEM((R * C, G), jnp.float32),
            pltpu.VMEM((C2,), jnp.float32),
            pltpu.VMEM((C2,), jnp.int32),
            pltpu.VMEM((R * TOP_K,), jnp.int32),
            pltpu.VMEM((R * TOP_K, DIM), jnp.float32),
            pltpu.VMEM((R * TOP_K,), jnp.float32),
            pltpu.SemaphoreType.DMA,
            pltpu.SemaphoreType.DMA,
            pltpu.SemaphoreType.DMA,
        ],
        compiler_params=pltpu.CompilerParams(needs_layout_passes=False),
    )
    def sc_kernel(simc_hbm, gmax_hbm, memory_hbm,
                  tsim_hbm, rows_hbm,
                  gmax_a, gmax_b, idx_v, cand_v, c2v, c2i, slot_v, rows_v,
                  ts_v, sem_g, sem_c, sem_r):
        gmax_bufs = (gmax_a, gmax_b)
        wid = lax.axis_index("s") * NC + lax.axis_index("c")
        row0 = wid * ROWS_PER_W

        pltpu.make_async_copy(
            gmax_hbm.at[pl.ds(row0 * NG, GSTRIDE)], gmax_a, sem_g).start()

        def chunk_body(chunk, b):
            grow0 = row0 + chunk * R
            pltpu.make_async_copy(
                gmax_hbm.at[pl.ds(0, GSTRIDE)], gmax_bufs[b], sem_g).wait()
            nxt = row0 + jnp.minimum(chunk + 1, NCHW - 1) * R
            pltpu.make_async_copy(
                gmax_hbm.at[pl.ds(nxt * NG, GSTRIDE)],
                gmax_bufs[1 - b], sem_g).start()

            def s1_row(rr, ths):
                nv = NG // 16
                sec = []
                for s0 in range(0, nv, 17):
                    m = gmax_bufs[b][pl.ds(rr * NG + s0 * 16, 16)]
                    for j in range(s0 + 1, min(s0 + 17, nv)):
                        m = jnp.maximum(
                            m, gmax_bufs[b][pl.ds(rr * NG + j * 16, 16)])
                    sec.append(m)
                mlo = jnp.minimum(jnp.minimum(sec[0], sec[1]), sec[2])
                mhi = jnp.maximum(jnp.maximum(sec[0], sec[1]), sec[2])
                slo, _ = plsc.sort_key_val(mlo, _iota16(), descending=True)
                shi, _ = plsc.sort_key_val(mhi, _iota16(), descending=True)
                lo = _lane_bcast(slo, 15) - 1e-6
                hi = _lane_bcast(shi, 0)

                def bis_body(i, lh):
                    lo, hi = lh
                    mid = 0.5 * (lo + hi)
                    acc = jnp.zeros((16,), jnp.int32)
                    for j in range(nv):
                        v = gmax_bufs[b][pl.ds(rr * NG + j * 16, 16)]
                        acc = acc + plsc.all_reduce_population_count(v > mid)
                    big = acc[0] >= TOP_K
                    return (jnp.where(big, mid, lo), jnp.where(big, hi, mid))

                th, _ = lax.fori_loop(0, BIS_IT, bis_body, (lo, hi))

                pad_chunk = jnp.full((16,), (grow0 + rr) * NG + NG - 1,
                                     jnp.int32)
                for q in range(C // 16):
                    idx_v[pl.ds(rr * C + q * 16, 16)] = pad_chunk

                def col_body(j, cur):
                    v = gmax_bufs[b][pl.ds(rr * NG + j * 16, 16)]
                    m = v > th
                    gid = j * 16 + _iota16() + (grow0 + rr) * NG
                    plsc.store_compressed(idx_v.at[pl.ds(cur, 16)], gid,
                                          mask=m)
                    cnt = plsc.all_reduce_population_count(m)[0]
                    return jnp.minimum(cur + cnt, rr * C + (C - 16))

                lax.fori_loop(0, NG // 16, col_body, rr * C)
                return ths + (th,)

            ths = ()
            for rr in range(R):
                ths = s1_row(rr, ths)

            pltpu.make_async_copy(simc_hbm.at[idx_v], cand_v, sem_c).start()
            pltpu.make_async_copy(simc_hbm.at[idx_v], cand_v, sem_c).wait()

            for rr in range(R):
                th = ths[rr]
                neg = jnp.full((16,), INIT_VAL, jnp.float32)
                for q in range(C2 // 16):
                    c2v[pl.ds(q * 16, 16)] = neg

                def scan_body(t, cur, rr=rr, th=th):
                    civ16 = idx_v[pl.ds(rr * C + (t >> 4) * 16, 16)]
                    civ = _lane_bcast(civ16, t & 15)
                    colbase = (civ - (grow0 + rr) * NG) * G
                    for j in range(G // 16):
                        v = cand_v[rr * C + t, pl.ds(j * 16, 16)]
                        m = v > th
                        col = colbase + j * 16 + _iota16()
                        plsc.store_compressed(c2v.at[pl.ds(cur, 16)], v,
                                              mask=m)
                        plsc.store_compressed(c2i.at[pl.ds(cur, 16)], col,
                                              mask=m)
                        cnt = plsc.all_reduce_population_count(m)[0]
                        cur = jnp.minimum(cur + cnt, C2 - 16)
                    return cur

                lax.fori_loop(0, C, scan_body, 0)

                state = _init_state()
                for q in range(C2 // 16):
                    state = _merge_top32(*state,
                                         c2v[pl.ds(q * 16, 16)],
                                         c2i[pl.ds(q * 16, 16)])
                v0, x0, v1, x1 = state
                ts_v[pl.ds(rr * TOP_K, 16)] = v0
                ts_v[pl.ds(rr * TOP_K + 16, 16)] = v1
                slot_v[pl.ds(rr * TOP_K, 16)] = x0
                slot_v[pl.ds(rr * TOP_K + 16, 16)] = x1

            pltpu.make_async_copy(memory_hbm.at[slot_v], rows_v, sem_r).start()
            pltpu.sync_copy(ts_v, tsim_hbm.at[pl.ds(grow0 * TOP_K, R * TOP_K)])
            pltpu.make_async_copy(memory_hbm.at[slot_v], rows_v, sem_r).wait()
            pltpu.sync_copy(rows_v,
                            rows_hbm.at[pl.ds(grow0 * TOP_K, R * TOP_K)])

        def cc_body(cc, _):
            chunk_body(cc * 2, 0)
            chunk_body(cc * 2 + 1, 1)
            return 0

        lax.fori_loop(0, NCHW // 2, cc_body, 0)
        pltpu.make_async_copy(
            gmax_hbm.at[pl.ds(0, GSTRIDE)], gmax_a, sem_g).wait()

    return sc_kernel


_sc_kernel = _make_sc_kernel()


@jax.jit
def kernel(query, memory, W):
    qn = pl.pallas_call(
        _qn_kernel,
        grid=(BATCH // BQ,),
        in_specs=[
            pl.BlockSpec((BQ, DIM), lambda i: (i, 0)),
            pl.BlockSpec((DIM, DIM), lambda i: (0, 0)),
        ],
        out_specs=pl.BlockSpec((BQ, DIM), lambda i: (i, 0)),
        out_shape=jax.ShapeDtypeStruct((BATCH, DIM), jnp.float32),
    )(query, W)

    mem_pad = jnp.pad(memory, ((0, NPAD - MEM_SIZE), (0, 0)))

    sim, gmax3 = pl.pallas_call(
        _sim_kernel,
        grid=(BATCH // BQ, NPAD // BN),
        in_specs=[
            pl.BlockSpec((BQ, DIM), lambda i, j: (i, 0)),
            pl.BlockSpec((BN, DIM), lambda i, j: (j, 0)),
        ],
        out_specs=[
            pl.BlockSpec((BQ, BN), lambda i, j: (i, j)),
            pl.BlockSpec((1, BQ, NGJ), lambda i, j: (j, i, 0)),
        ],
        out_shape=[
            jax.ShapeDtypeStruct((BATCH, NPAD), jnp.float32),
            jax.ShapeDtypeStruct((NPAD // BN, BATCH, NGJ), jnp.float32),
        ],
    )(qn, mem_pad)

    gmax = pl.pallas_call(
        _gmax_t_kernel,
        grid=(BATCH // BQ,),
        in_specs=[pl.BlockSpec((NPAD // BN, BQ, NGJ), lambda i: (0, i, 0))],
        out_specs=pl.BlockSpec((BQ, NG), lambda i: (i, 0)),
        out_shape=jax.ShapeDtypeStruct((BATCH, NG), jnp.float32),
    )(gmax3)

    sim_chunks = sim.reshape(BATCH * NG, G)

    gmax_flat = jnp.pad(gmax.reshape(-1), (0, 128))
    top_sim_f, rows_f = _sc_kernel(sim_chunks, gmax_flat, memory)
    top_sim = top_sim_f.reshape(BATCH, TOP_K)
    rows = rows_f.reshape(BATCH, TOP_K, DIM)

    retrieved = pl.pallas_call(
        _combine_kernel,
        grid=(BATCH // BQ,),
        in_specs=[
            pl.BlockSpec((BQ, TOP_K), lambda i: (i, 0)),
            pl.BlockSpec((BQ, TOP_K, DIM), lambda i: (i, 0, 0)),
        ],
        out_specs=pl.BlockSpec((BQ, DIM), lambda i: (i, 0)),
        out_shape=jax.ShapeDtypeStruct((BATCH, DIM), jnp.float32),
    )(top_sim, rows)

    return retrieved, top_sim


# speedup vs baseline: 10.7638x; 1.0634x over previous
import functools

import jax
import jax.numpy as jnp
from jax import lax
from jax.experimental import pallas as pl
from jax.experimental.pallas import tpu as pltpu
from jax.experimental.pallas import tpu_sc as plsc

MEM_SIZE = 100000
DIM = 256
TOP_K = 32
BATCH = 4096

BQ = 512
BN = 2048
NPAD = 102400
G = 128
NG = NPAD // G
NGJ = BN // G
PAD_VAL = -2.0
INIT_VAL = -3.0

NC = 2
NS = 16
NW = NC * NS
ROWS_PER_W = BATCH // NW


def _qn_kernel(q_ref, w_ref, qn_ref):
    q = q_ref[...]
    w = w_ref[...]
    qp = lax.dot_general(q, w, (((1,), (1,)), ((), ())),
                         preferred_element_type=jnp.float32)
    norm = jnp.sqrt(jnp.sum(qp * qp, axis=-1, keepdims=True))
    qn_ref[...] = qp / jnp.maximum(norm, 1e-12)


def _sim_kernel(qn_ref, mem_ref, sim_ref, gmax_ref):
    j = pl.program_id(1)
    sim = lax.dot_general(qn_ref[...], mem_ref[...], (((1,), (1,)), ((), ())),
                          preferred_element_type=jnp.float32)
    col = j * BN + lax.broadcasted_iota(jnp.int32, (BQ, BN), 1)
    sim = jnp.where(col < MEM_SIZE, sim, PAD_VAL)
    sim_ref[...] = sim
    gmax_ref[...] = jnp.max(sim.reshape(BQ, NGJ, G), axis=2)[None]


def _gmax_t_kernel(g3_ref, g2_ref):
    x = g3_ref[...]
    g2_ref[...] = jnp.transpose(x, (1, 0, 2)).reshape(BQ, NG)


def _combine_kernel(ts_ref, rows_ref, out_ref):
    ts = ts_ref[...]
    rows = rows_ref[...]
    m = jnp.max(ts, axis=-1, keepdims=True)
    e = jnp.exp(ts - m)
    w = e / jnp.sum(e, axis=-1, keepdims=True)
    acc = jnp.zeros((BQ, DIM), jnp.float32)
    for k in range(TOP_K):
        acc = acc + w[:, k][:, None] * rows[:, k, :]
    out_ref[...] = acc


def _iota16():
    return lax.iota(jnp.int32, 16)


_BCAST_DNUMS = lax.GatherDimensionNumbers(
    offset_dims=(), collapsed_slice_dims=(0,), start_index_map=(0,))


def _lane_bcast(vec, lane):
    idx = jnp.full((16, 1), lane, jnp.int32)
    return lax.gather(vec, idx, _BCAST_DNUMS, (1,),
                      mode=lax.GatherScatterMode.PROMISE_IN_BOUNDS)


def _merge_top32(v0, x0, v1, x1, vm, ix):
    s, si = plsc.sort_key_val(vm, ix, descending=True)
    rs = lax.rev(s, (0,))
    rsi = lax.rev(si, (0,))
    take = v0 >= rs
    hi = jnp.where(take, v0, rs)
    hix = jnp.where(take, x0, rsi)
    lo = jnp.where(take, rs, v0)
    lox = jnp.where(take, rsi, x0)
    v0n, x0n = plsc.sort_key_val(hi, hix, descending=True)
    ls, lsi = plsc.sort_key_val(lo, lox, descending=True)
    rls = lax.rev(ls, (0,))
    rlsi = lax.rev(lsi, (0,))
    take2 = v1 >= rls
    hi2 = jnp.where(take2, v1, rls)
    hix2 = jnp.where(take2, x1, rlsi)
    v1n, x1n = plsc.sort_key_val(hi2, hix2, descending=True)
    return v0n, x0n, v1n, x1n


def _maybe_merge(state, v, ix, th0=None):
    v0, x0, v1, x1 = state
    m = v > _lane_bcast(v1, 15)
    if th0 is not None:
        m = m | (v >= th0)
    return lax.cond(
        jnp.any(m),
        lambda: _merge_top32(v0, x0, v1, x1, jnp.where(m, v, INIT_VAL), ix),
        lambda: state,
    )


def _init_state():
    z = jnp.full((16,), INIT_VAL, jnp.float32)
    zi = jnp.zeros((16,), jnp.int32)
    return (z, zi, z, zi)


R = 2
C = 64
C2 = 160
BIS_IT = 12
NCHW = ROWS_PER_W // R
GSTRIDE = R * NG + 64


def _make_sc_kernel(nbatch):
    rows_per_w = nbatch // NW
    nchw = rows_per_w // R
    mesh = plsc.VectorSubcoreMesh(core_axis_name="c", subcore_axis_name="s")

    @functools.partial(
        pl.kernel,
        mesh=mesh,
        out_type=[
            jax.ShapeDtypeStruct((nbatch * TOP_K,), jnp.float32),
            jax.ShapeDtypeStruct((nbatch * TOP_K, DIM), jnp.float32),
        ],
        scratch_types=[
            pltpu.VMEM((GSTRIDE,), jnp.float32),
            pltpu.VMEM((GSTRIDE,), jnp.float32),
            pltpu.VMEM((R * C,), jnp.int32),
            pltpu.VMEM((R * C, G), jnp.float32),
            pltpu.VMEM((C2,), jnp.float32),
            pltpu.VMEM((C2,), jnp.int32),
            pltpu.VMEM((R * TOP_K,), jnp.int32),
            pltpu.VMEM((R * TOP_K, DIM), jnp.float32),
            pltpu.VMEM((R * TOP_K,), jnp.float32),
            pltpu.SemaphoreType.DMA,
            pltpu.SemaphoreType.DMA,
            pltpu.SemaphoreType.DMA,
        ],
        compiler_params=pltpu.CompilerParams(needs_layout_passes=False),
    )
    def sc_kernel(simc_hbm, gmax_hbm, memory_hbm,
                  tsim_hbm, rows_hbm,
                  gmax_a, gmax_b, idx_v, cand_v, c2v, c2i, slot_v, rows_v,
                  ts_v, sem_g, sem_c, sem_r):
        gmax_bufs = (gmax_a, gmax_b)
        wid = lax.axis_index("s") * NC + lax.axis_index("c")
        row0 = wid * rows_per_w

        pltpu.make_async_copy(
            gmax_hbm.at[pl.ds(row0 * NG, GSTRIDE)], gmax_a, sem_g).start()

        def chunk_body(chunk, b):
            grow0 = row0 + chunk * R
            pltpu.make_async_copy(
                gmax_hbm.at[pl.ds(0, GSTRIDE)], gmax_bufs[b], sem_g).wait()
            nxt = row0 + jnp.minimum(chunk + 1, nchw - 1) * R
            pltpu.make_async_copy(
                gmax_hbm.at[pl.ds(nxt * NG, GSTRIDE)],
                gmax_bufs[1 - b], sem_g).start()

            def s1_row(rr, ths):
                nv = NG // 16
                sec = []
                for s0 in range(0, nv, 17):
                    m = gmax_bufs[b][pl.ds(rr * NG + s0 * 16, 16)]
                    for j in range(s0 + 1, min(s0 + 17, nv)):
                        m = jnp.maximum(
                            m, gmax_bufs[b][pl.ds(rr * NG + j * 16, 16)])
                    sec.append(m)
                mlo = jnp.minimum(jnp.minimum(sec[0], sec[1]), sec[2])
                mhi = jnp.maximum(jnp.maximum(sec[0], sec[1]), sec[2])
                slo, _ = plsc.sort_key_val(mlo, _iota16(), descending=True)
                shi, _ = plsc.sort_key_val(mhi, _iota16(), descending=True)
                lo = _lane_bcast(slo, 15) - 1e-6
                hi = _lane_bcast(shi, 0)

                def bis_body(i, lh):
                    lo, hi = lh
                    mid = 0.5 * (lo + hi)
                    acc = jnp.zeros((16,), jnp.int32)
                    for j in range(nv):
                        v = gmax_bufs[b][pl.ds(rr * NG + j * 16, 16)]
                        acc = acc + plsc.all_reduce_population_count(v > mid)
                    big = acc[0] >= TOP_K
                    return (jnp.where(big, mid, lo), jnp.where(big, hi, mid))

                th, _ = lax.fori_loop(0, BIS_IT, bis_body, (lo, hi))

                pad_chunk = jnp.full((16,), (grow0 + rr) * NG + NG - 1,
                                     jnp.int32)
                for q in range(C // 16):
                    idx_v[pl.ds(rr * C + q * 16, 16)] = pad_chunk

                def col_body(j, cur):
                    v = gmax_bufs[b][pl.ds(rr * NG + j * 16, 16)]
                    m = v > th
                    gid = j * 16 + _iota16() + (grow0 + rr) * NG
                    plsc.store_compressed(idx_v.at[pl.ds(cur, 16)], gid,
                                          mask=m)
                    cnt = plsc.all_reduce_population_count(m)[0]
                    return jnp.minimum(cur + cnt, rr * C + (C - 16))

                lax.fori_loop(0, NG // 16, col_body, rr * C)
                return ths + (th,)

            ths = ()
            for rr in range(R):
                ths = s1_row(rr, ths)

            pltpu.make_async_copy(simc_hbm.at[idx_v], cand_v, sem_c).start()
            pltpu.make_async_copy(simc_hbm.at[idx_v], cand_v, sem_c).wait()

            for rr in range(R):
                th = ths[rr]
                neg = jnp.full((16,), INIT_VAL, jnp.float32)
                for q in range(C2 // 16):
                    c2v[pl.ds(q * 16, 16)] = neg

                def scan_body(t, cur, rr=rr, th=th):
                    civ16 = idx_v[pl.ds(rr * C + (t >> 4) * 16, 16)]
                    civ = _lane_bcast(civ16, t & 15)
                    colbase = (civ - (grow0 + rr) * NG) * G
                    for j in range(G // 16):
                        v = cand_v[rr * C + t, pl.ds(j * 16, 16)]
                        m = v > th
                        col = colbase + j * 16 + _iota16()
                        plsc.store_compressed(c2v.at[pl.ds(cur, 16)], v,
                                              mask=m)
                        plsc.store_compressed(c2i.at[pl.ds(cur, 16)], col,
                                              mask=m)
                        cnt = plsc.all_reduce_population_count(m)[0]
                        cur = jnp.minimum(cur + cnt, C2 - 16)
                    return cur

                lax.fori_loop(0, C, scan_body, 0)

                state = _init_state()
                for q in range(C2 // 16):
                    state = _merge_top32(*state,
                                         c2v[pl.ds(q * 16, 16)],
                                         c2i[pl.ds(q * 16, 16)])
                v0, x0, v1, x1 = state
                ts_v[pl.ds(rr * TOP_K, 16)] = v0
                ts_v[pl.ds(rr * TOP_K + 16, 16)] = v1
                slot_v[pl.ds(rr * TOP_K, 16)] = x0
                slot_v[pl.ds(rr * TOP_K + 16, 16)] = x1

            pltpu.make_async_copy(memory_hbm.at[slot_v], rows_v, sem_r).start()
            pltpu.sync_copy(ts_v, tsim_hbm.at[pl.ds(grow0 * TOP_K, R * TOP_K)])
            pltpu.make_async_copy(memory_hbm.at[slot_v], rows_v, sem_r).wait()
            pltpu.sync_copy(rows_v,
                            rows_hbm.at[pl.ds(grow0 * TOP_K, R * TOP_K)])

        def cc_body(cc, _):
            chunk_body(cc * 2, 0)
            chunk_body(cc * 2 + 1, 1)
            return 0

        lax.fori_loop(0, nchw // 2, cc_body, 0)
        pltpu.make_async_copy(
            gmax_hbm.at[pl.ds(0, GSTRIDE)], gmax_a, sem_g).wait()

    return sc_kernel


_sc_kernel = _make_sc_kernel(BATCH // 2)


HB = BATCH // 2


def _sim_half(qn_h, mem_pad):
    sim, gmax3 = pl.pallas_call(
        _sim_kernel,
        grid=(HB // BQ, NPAD // BN),
        in_specs=[
            pl.BlockSpec((BQ, DIM), lambda i, j: (i, 0)),
            pl.BlockSpec((BN, DIM), lambda i, j: (j, 0)),
        ],
        out_specs=[
            pl.BlockSpec((BQ, BN), lambda i, j: (i, j)),
            pl.BlockSpec((1, BQ, NGJ), lambda i, j: (j, i, 0)),
        ],
        out_shape=[
            jax.ShapeDtypeStruct((HB, NPAD), jnp.float32),
            jax.ShapeDtypeStruct((NPAD // BN, HB, NGJ), jnp.float32),
        ],
    )(qn_h, mem_pad)

    gmax = pl.pallas_call(
        _gmax_t_kernel,
        grid=(HB // BQ,),
        in_specs=[pl.BlockSpec((NPAD // BN, BQ, NGJ), lambda i: (0, i, 0))],
        out_specs=pl.BlockSpec((BQ, NG), lambda i: (i, 0)),
        out_shape=jax.ShapeDtypeStruct((HB, NG), jnp.float32),
    )(gmax3)
    return sim, gmax


@jax.jit
def kernel(query, memory, W):
    qn = pl.pallas_call(
        _qn_kernel,
        grid=(BATCH // BQ,),
        in_specs=[
            pl.BlockSpec((BQ, DIM), lambda i: (i, 0)),
            pl.BlockSpec((DIM, DIM), lambda i: (0, 0)),
        ],
        out_specs=pl.BlockSpec((BQ, DIM), lambda i: (i, 0)),
        out_shape=jax.ShapeDtypeStruct((BATCH, DIM), jnp.float32),
    )(query, W)

    mem_pad = jnp.pad(memory, ((0, NPAD - MEM_SIZE), (0, 0)))

    sim0, gmax0 = _sim_half(qn[:HB], mem_pad)
    sim1, gmax1 = _sim_half(qn[HB:], mem_pad)

    halves = []
    for sim, gmax in ((sim0, gmax0), (sim1, gmax1)):
        sim_chunks = sim.reshape(HB * NG, G)
        gmax_flat = jnp.pad(gmax.reshape(-1), (0, 128))
        ts_f, rows_f = _sc_kernel(sim_chunks, gmax_flat, memory)
        halves.append((ts_f.reshape(HB, TOP_K),
                       rows_f.reshape(HB, TOP_K, DIM)))

    top_sim = jnp.concatenate([h[0] for h in halves], axis=0)
    rows = jnp.concatenate([h[1] for h in halves], axis=0)

    retrieved = pl.pallas_call(
        _combine_kernel,
        grid=(BATCH // BQ,),
        in_specs=[
            pl.BlockSpec((BQ, TOP_K), lambda i: (i, 0)),
            pl.BlockSpec((BQ, TOP_K, DIM), lambda i: (i, 0, 0)),
        ],
        out_specs=pl.BlockSpec((BQ, DIM), lambda i: (i, 0)),
        out_shape=jax.ShapeDtypeStruct((BATCH, DIM), jnp.float32),
    )(top_sim, rows)

    return retrieved, top_sim
